# baseline (speedup 1.0000x reference)
import functools

import jax
import jax.numpy as jnp
from jax import lax
from jax.experimental import pallas as pl
from jax.experimental.pallas import tpu as pltpu
from jax.experimental.pallas import tpu_sc as plsc

_B, _S, _D = 4, 4096, 1024
_NC, _NS, _L = 2, 16, 16
_NW = _NC * _NS
_SPW = _S // _NW
_CS = 16
_NCHUNK = _SPW // _CS
_CE = _CS * _D
_NBUF = 5
_NITEMS = _NCHUNK * _B
_PD = 3


def _make_sc_kernel():
    mesh = plsc.VectorSubcoreMesh(core_axis_name="c", subcore_axis_name="s")

    @functools.partial(
        pl.kernel,
        out_type=jax.ShapeDtypeStruct((_B * _S * _D,), jnp.float32),
        mesh=mesh,
        scratch_types=[pltpu.VMEM_SHARED((_NS * _NBUF * _CE,), jnp.float32)]
        + [pltpu.SemaphoreType.DMA] * (2 * _NBUF),
    )
    def sc_copy(x_hbm, out_hbm, shared, *sems):
        in_sems = sems[:_NBUF]
        out_sems = sems[_NBUF:]

        sid = lax.axis_index("s")
        wid = sid * _NC + lax.axis_index("c")
        base = wid * (_SPW * _D)
        sbase = sid * (_NBUF * _CE)

        def x_off(k):
            c, b = divmod(k, _B)
            return b * (_S * _D) + base + c * _CE

        def buf(j):
            return shared.at[pl.ds(sbase + j * _CE, _CE)]

        def gather_x(k):
            j = k % _NBUF
            return pltpu.async_copy(
                x_hbm.at[pl.ds(x_off(k), _CE)], buf(j), in_sems[j]
            )

        x_d = {k: gather_x(k) for k in range(_PD)}
        scat_d = {}

        for k in range(_NITEMS):
            j = k % _NBUF
            x_d[k].wait()
            scat_d[k] = pltpu.async_copy(
                buf(j), out_hbm.at[pl.ds(x_off(k), _CE)], out_sems[j]
            )
            kn = k + _PD
            if kn < _NITEMS:
                if kn - _NBUF >= 0:
                    scat_d[kn - _NBUF].wait()
                x_d[kn] = gather_x(kn)

        for k in range(_NITEMS - _NBUF, _NITEMS):
            if k >= 0:
                scat_d[k].wait()

    return sc_copy


_sc_copy = _make_sc_kernel()


def kernel(x, pos_table):
    b, s, d = x.shape
    out = _sc_copy(x.reshape(-1))
    return out.reshape(b, s, d)


# speedup vs baseline: 1.0027x; 1.0027x over previous
import functools

import jax
import jax.numpy as jnp
from jax import lax
from jax.experimental import pallas as pl
from jax.experimental.pallas import tpu as pltpu
from jax.experimental.pallas import tpu_sc as plsc

_B, _S, _D = 4, 4096, 1024
_NC, _NS, _L = 2, 16, 16
_NW = _NC * _NS
_SPW = _S // _NW
_CS = 32
_NCHUNK = _SPW // _CS
_CE = _CS * _D
_NBUF = 3
_NITEMS = _NCHUNK * _B
_PD = 2


def _make_sc_kernel():
    mesh = plsc.VectorSubcoreMesh(core_axis_name="c", subcore_axis_name="s")

    @functools.partial(
        pl.kernel,
        out_type=jax.ShapeDtypeStruct((_B * _S * _D,), jnp.float32),
        mesh=mesh,
        scratch_types=[pltpu.VMEM_SHARED((_NS * _NBUF * _CE,), jnp.float32)]
        + [pltpu.SemaphoreType.DMA] * (2 * _NBUF),
    )
    def sc_copy(x_hbm, out_hbm, shared, *sems):
        in_sems = sems[:_NBUF]
        out_sems = sems[_NBUF:]

        sid = lax.axis_index("s")
        wid = sid * _NC + lax.axis_index("c")
        base = wid * (_SPW * _D)
        sbase = sid * (_NBUF * _CE)

        def x_off(k):
            c, b = divmod(k, _B)
            return b * (_S * _D) + base + c * _CE

        def buf(j):
            return shared.at[pl.ds(sbase + j * _CE, _CE)]

        def gather_x(k):
            j = k % _NBUF
            return pltpu.async_copy(
                x_hbm.at[pl.ds(x_off(k), _CE)], buf(j), in_sems[j]
            )

        x_d = {k: gather_x(k) for k in range(_PD)}
        scat_d = {}

        for k in range(_NITEMS):
            j = k % _NBUF
            x_d[k].wait()
            scat_d[k] = pltpu.async_copy(
                buf(j), out_hbm.at[pl.ds(x_off(k), _CE)], out_sems[j]
            )
            kn = k + _PD
            if kn < _NITEMS:
                if kn - _NBUF >= 0:
                    scat_d[kn - _NBUF].wait()
                x_d[kn] = gather_x(kn)

        for k in range(_NITEMS - _NBUF, _NITEMS):
            if k >= 0:
                scat_d[k].wait()

    return sc_copy


_sc_copy = _make_sc_kernel()


def kernel(x, pos_table):
    b, s, d = x.shape
    out = _sc_copy(x.reshape(-1))
    return out.reshape(b, s, d)


# speedup vs baseline: 3.9128x; 3.9021x over previous
import jax
import jax.numpy as jnp
from jax.experimental import pallas as pl

_BS = 2048


def _body(x_ref, pos_ref, o_ref):
    o_ref[...] = x_ref[...] + pos_ref[...][None, :, :]


def kernel(x, pos_table):
    b, s, d = x.shape
    return pl.pallas_call(
        _body,
        grid=(s // _BS, b),
        in_specs=[
            pl.BlockSpec((1, _BS, d), lambda i, j: (j, i, 0)),
            pl.BlockSpec((_BS, d), lambda i, j: (i, 0)),
        ],
        out_specs=pl.BlockSpec((1, _BS, d), lambda i, j: (j, i, 0)),
        out_shape=jax.ShapeDtypeStruct((b, s, d), x.dtype),
    )(x, pos_table)
